# baseline (speedup 1.0000x reference)
import functools

import jax
import jax.numpy as jnp
from jax import lax
from jax.experimental import pallas as pl
from jax.experimental.pallas import tpu as pltpu
from jax.experimental.pallas import tpu_sc as plsc

_NUM_CORES = 2
_NUM_SUBCORES = 16
_NW = _NUM_CORES * _NUM_SUBCORES
_L = 16

_PARAMS = pltpu.CompilerParams(
    use_tc_tiling_on_sc=True,
    skip_device_barrier=True,
    needs_layout_passes=False,
)


def _iota():
    return lax.iota(jnp.int32, _L)


def _diag(q):
    return lax.bitwise_and(_iota() + q, _L - 1)


def _relayout_kernel(V, D):
    n_blocks = V // 128
    n_pairs = V // 2
    tail_pairs = (V - n_blocks * 128) // 2
    iters = -(-n_blocks // _NW)
    mesh = plsc.VectorSubcoreMesh(core_axis_name="c", subcore_axis_name="s")

    @functools.partial(
        pl.kernel,
        mesh=mesh,
        compiler_params=_PARAMS,
        out_type=jax.ShapeDtypeStruct((n_pairs, 2 * D), jnp.float32),
        scratch_types=[
            pltpu.VMEM((2 * D, 128), jnp.float32),
            pltpu.VMEM((D, 2 * D), jnp.float32),
            pltpu.VMEM((tail_pairs, 2 * D), jnp.float32),
            pltpu.SemaphoreType.DMA,
            pltpu.SemaphoreType.DMA,
        ],
    )
    def k(tabT_hbm, tail_hbm, pairs_hbm, blk_v, p_v, tail_v, sem_rd, sem_wr):
        wid = lax.axis_index("s") * _NUM_CORES + lax.axis_index("c")

        @pl.when(wid == _NW - 1)
        def _():
            pltpu.sync_copy(tail_hbm, tail_v)
            pltpu.sync_copy(tail_v, pairs_hbm.at[pl.ds(n_blocks * D, tail_pairs)])

        def start_read(bank, g):
            pltpu.async_copy(
                tabT_hbm.at[:, pl.ds(g * 128, 128)],
                blk_v.at[pl.ds(bank * D, D)],
                sem_rd,
            )

        def wait_read():
            pltpu.make_async_copy(
                tabT_hbm.at[:, pl.ds(0, 128)], blk_v.at[pl.ds(0, D)], sem_rd
            ).wait()

        def wait_write():
            pltpu.make_async_copy(
                p_v, pairs_hbm.at[pl.ds(0, D)], sem_wr
            ).wait()

        start_read(0, wid)

        @pl.loop(0, iters)
        def _(i):
            g = i * _NW + wid
            bank_off = (i % 2) * D
            nxt = g + _NW

            @pl.when(g < n_blocks)
            def _():
                wait_read()

                @pl.when(nxt < n_blocks)
                def _():
                    start_read((i + 1) % 2, nxt)

                @pl.when(i > 0)
                def _():
                    wait_write()

                @pl.loop(0, 128 // _L)
                def _(cg):
                    cg16 = cg * _L
                    cg8 = cg * 8
                    for dg in range(D // _L):
                        dgoff = dg * _L + bank_off
                        dvec = _iota() + dgoff
                        for qb in range(2):
                            rng = range(qb * 8, qb * 8 + 8)
                            vals = [
                                plsc.load_gather(blk_v, [dvec, _diag(q) + cg16])
                                for q in rng
                            ]
                            for qi, q in enumerate(rng):
                                rowv = lax.shift_right_logical(_diag(q), 1) + cg8
                                colv = (
                                    lax.bitwise_and(_diag(q), 1) * D + _iota()
                                ) + dgoff
                                plsc.store_scatter(p_v, [rowv, colv], vals[qi])
                pltpu.async_copy(p_v, pairs_hbm.at[pl.ds(g * D, D)], sem_wr)

        wait_write()

    return k


def _gather_kernel(B, V, D):
    b_per_w = B // _NW
    chunk = 64
    n_chunks = b_per_w // chunk
    mesh = plsc.VectorSubcoreMesh(core_axis_name="c", subcore_axis_name="s")

    @functools.partial(
        pl.kernel,
        mesh=mesh,
        compiler_params=_PARAMS,
        out_type=jax.ShapeDtypeStruct((D, B), jnp.float32),
        scratch_types=[
            pltpu.VMEM((b_per_w,), jnp.int32),
            pltpu.VMEM((b_per_w,), jnp.int32),
            pltpu.VMEM((2 * chunk, 2 * D), jnp.float32),
            pltpu.VMEM((D, b_per_w), jnp.float32),
            pltpu.SemaphoreType.DMA,
        ],
    )
    def k(idx_hbm, pairs_hbm, out_hbm, idx_v, pidx_v, buf_v, outT_v, sem):
        wid = lax.axis_index("s") * _NUM_CORES + lax.axis_index("c")
        base = wid * b_per_w
        pltpu.sync_copy(idx_hbm.at[pl.ds(base, b_per_w)], idx_v)
        for q in range(b_per_w // _L):
            s = pl.ds(q * _L, _L)
            pidx_v[s] = lax.shift_right_logical(idx_v[s], 1)

        def fire(c):
            pltpu.async_copy(
                pairs_hbm.at[pidx_v.at[pl.ds(c * chunk, chunk)]],
                buf_v.at[pl.ds((c % 2) * chunk, chunk)],
                sem,
            )

        def wait_gather():
            pltpu.make_async_copy(
                pairs_hbm.at[pidx_v.at[pl.ds(0, chunk)]],
                buf_v.at[pl.ds(0, chunk)],
                sem,
            ).wait()

        fire(0)

        @pl.loop(0, n_chunks)
        def _(c):
            wait_gather()

            @pl.when(c + 1 < n_chunks)
            def _():
                fire(c + 1)

            @pl.loop(0, chunk // _L)
            def _(ig):
                halfoff = lax.bitwise_and(
                    idx_v[pl.ds(c * chunk + ig * _L, _L)], 1) * D
                lidx = _iota() + ((c % 2) * chunk + ig * _L)
                idpos = _iota() + (c * chunk + ig * _L)
                for cg in range(D // _L):
                    for qb in range(2):
                        vals = []
                        for q in range(qb * 8, qb * 8 + 8):
                            cvec = _diag(q) + cg * _L
                            vals.append(
                                plsc.load_gather(buf_v, [lidx, halfoff + cvec])
                            )
                        for qi, q in enumerate(range(qb * 8, qb * 8 + 8)):
                            cvec = _diag(q) + cg * _L
                            plsc.store_scatter(outT_v, [cvec, idpos], vals[qi])
        pltpu.sync_copy(outT_v, out_hbm.at[:, pl.ds(base, b_per_w)])

    return k


def kernel(dataset_ids, prompt_embedding):
    B = dataset_ids.shape[0]
    V, D = prompt_embedding.shape
    n_blocks = V // 128
    tabT = prompt_embedding.T
    tail = prompt_embedding[n_blocks * 128:].reshape(-1, 2 * D)
    pairs = _relayout_kernel(V, D)(tabT, tail)
    outT = _gather_kernel(B, V, D)(dataset_ids.astype(jnp.int32), pairs)
    return outT.T


# speedup vs baseline: 1.0021x; 1.0021x over previous
import functools

import jax
import jax.numpy as jnp
from jax import lax
from jax.experimental import pallas as pl
from jax.experimental.pallas import tpu as pltpu
from jax.experimental.pallas import tpu_sc as plsc

_NUM_CORES = 2
_NUM_SUBCORES = 16
_NW = _NUM_CORES * _NUM_SUBCORES
_L = 16

_PARAMS = pltpu.CompilerParams(
    use_tc_tiling_on_sc=True,
    skip_device_barrier=True,
    needs_layout_passes=False,
)


def _iota():
    return lax.iota(jnp.int32, _L)


def _diag(q):
    return lax.bitwise_and(_iota() + q, _L - 1)


def _relayout_kernel(V, D):
    n_blocks = V // 128
    n_pairs = V // 2
    tail_pairs = (V - n_blocks * 128) // 2
    iters = -(-n_blocks // _NW)
    mesh = plsc.VectorSubcoreMesh(core_axis_name="c", subcore_axis_name="s")

    @functools.partial(
        pl.kernel,
        mesh=mesh,
        compiler_params=_PARAMS,
        out_type=jax.ShapeDtypeStruct((n_pairs, 2 * D), jnp.float32),
        scratch_types=[
            pltpu.VMEM((2 * D, 128), jnp.float32),
            pltpu.VMEM((D, 2 * D), jnp.float32),
            pltpu.VMEM((tail_pairs, 2 * D), jnp.float32),
            pltpu.SemaphoreType.DMA,
            pltpu.SemaphoreType.DMA,
        ],
    )
    def k(tabT_hbm, tail_hbm, pairs_hbm, blk_v, p_v, tail_v, sem_rd, sem_wr):
        wid = lax.axis_index("s") * _NUM_CORES + lax.axis_index("c")

        @pl.when(wid == _NW - 1)
        def _():
            pltpu.sync_copy(tail_hbm, tail_v)
            pltpu.sync_copy(tail_v, pairs_hbm.at[pl.ds(n_blocks * D, tail_pairs)])

        def start_read(bank, g):
            pltpu.async_copy(
                tabT_hbm.at[:, pl.ds(g * 128, 128)],
                blk_v.at[pl.ds(bank * D, D)],
                sem_rd,
            )

        def wait_read():
            pltpu.make_async_copy(
                tabT_hbm.at[:, pl.ds(0, 128)], blk_v.at[pl.ds(0, D)], sem_rd
            ).wait()

        def wait_write():
            pltpu.make_async_copy(
                p_v, pairs_hbm.at[pl.ds(0, D)], sem_wr
            ).wait()

        start_read(0, wid)

        @pl.loop(0, iters)
        def _(i):
            g = i * _NW + wid
            bank_off = (i % 2) * D
            nxt = g + _NW

            @pl.when(g < n_blocks)
            def _():
                wait_read()

                @pl.when(nxt < n_blocks)
                def _():
                    start_read((i + 1) % 2, nxt)

                @pl.when(i > 0)
                def _():
                    wait_write()

                @pl.loop(0, 128 // _L)
                def _(cg):
                    cg16 = cg * _L
                    cg8 = cg * 8
                    for dg in range(D // _L):
                        dgoff = dg * _L + bank_off
                        dvec = _iota() + dgoff
                        for qb in range(2):
                            rng = range(qb * 8, qb * 8 + 8)
                            vals = [
                                plsc.load_gather(blk_v, [dvec, _diag(q) + cg16])
                                for q in rng
                            ]
                            for qi, q in enumerate(rng):
                                rowv = lax.shift_right_logical(_diag(q), 1) + cg8
                                colv = (
                                    lax.bitwise_and(_diag(q), 1) * D + _iota()
                                ) + dg * _L
                                plsc.store_scatter(p_v, [rowv, colv], vals[qi])
                pltpu.async_copy(p_v, pairs_hbm.at[pl.ds(g * D, D)], sem_wr)

        wait_write()

    return k


def _gather_kernel(B, V, D):
    b_per_w = B // _NW
    chunk = 64
    n_chunks = b_per_w // chunk
    mesh = plsc.VectorSubcoreMesh(core_axis_name="c", subcore_axis_name="s")

    @functools.partial(
        pl.kernel,
        mesh=mesh,
        compiler_params=_PARAMS,
        out_type=jax.ShapeDtypeStruct((D, B), jnp.float32),
        scratch_types=[
            pltpu.VMEM((b_per_w,), jnp.int32),
            pltpu.VMEM((b_per_w,), jnp.int32),
            pltpu.VMEM((2 * chunk, 2 * D), jnp.float32),
            pltpu.VMEM((D, b_per_w), jnp.float32),
            pltpu.SemaphoreType.DMA,
        ],
    )
    def k(idx_hbm, pairs_hbm, out_hbm, idx_v, pidx_v, buf_v, outT_v, sem):
        wid = lax.axis_index("s") * _NUM_CORES + lax.axis_index("c")
        base = wid * b_per_w
        pltpu.sync_copy(idx_hbm.at[pl.ds(base, b_per_w)], idx_v)
        for q in range(b_per_w // _L):
            s = pl.ds(q * _L, _L)
            pidx_v[s] = lax.shift_right_logical(idx_v[s], 1)

        def fire(c):
            pltpu.async_copy(
                pairs_hbm.at[pidx_v.at[pl.ds(c * chunk, chunk)]],
                buf_v.at[pl.ds((c % 2) * chunk, chunk)],
                sem,
            )

        def wait_gather():
            pltpu.make_async_copy(
                pairs_hbm.at[pidx_v.at[pl.ds(0, chunk)]],
                buf_v.at[pl.ds(0, chunk)],
                sem,
            ).wait()

        fire(0)

        @pl.loop(0, n_chunks)
        def _(c):
            wait_gather()

            @pl.when(c + 1 < n_chunks)
            def _():
                fire(c + 1)

            @pl.loop(0, chunk // _L)
            def _(ig):
                halfoff = lax.bitwise_and(
                    idx_v[pl.ds(c * chunk + ig * _L, _L)], 1) * D
                lidx = _iota() + ((c % 2) * chunk + ig * _L)
                idpos = _iota() + (c * chunk + ig * _L)
                for cg in range(D // _L):
                    for qb in range(2):
                        vals = []
                        for q in range(qb * 8, qb * 8 + 8):
                            cvec = _diag(q) + cg * _L
                            vals.append(
                                plsc.load_gather(buf_v, [lidx, halfoff + cvec])
                            )
                        for qi, q in enumerate(range(qb * 8, qb * 8 + 8)):
                            cvec = _diag(q) + cg * _L
                            plsc.store_scatter(outT_v, [cvec, idpos], vals[qi])
        pltpu.sync_copy(outT_v, out_hbm.at[:, pl.ds(base, b_per_w)])

    return k


def kernel(dataset_ids, prompt_embedding):
    B = dataset_ids.shape[0]
    V, D = prompt_embedding.shape
    n_blocks = V // 128
    tabT = prompt_embedding.T
    tail = prompt_embedding[n_blocks * 128:].reshape(-1, 2 * D)
    pairs = _relayout_kernel(V, D)(tabT, tail)
    outT = _gather_kernel(B, V, D)(dataset_ids.astype(jnp.int32), pairs)
    return outT.T
